# Initial kernel scaffold; baseline (speedup 1.0000x reference)
import jax
import jax.numpy as jnp
from jax.experimental import pallas as pl


def kernel(data, neighbour, inv_neighbour, weights):
    raise NotImplementedError("write your pallas kernel here")


# speedup vs baseline: 1.1923x; 1.1923x over previous
import functools

import jax
import jax.numpy as jnp
from jax import lax
from jax.experimental import pallas as pl
from jax.experimental.pallas import tpu as pltpu
from jax.experimental.pallas import tpu_sc as plsc

_NW = 32
_CHUNK = 128


def _gather_body(nchunk, nj, idx_hbm, data_hbm, out_hbm, idx_v, rows_v, sem):
    wid = lax.axis_index("s") * 2 + lax.axis_index("c")

    def step(j, carry):
        c = wid + _NW * j

        @pl.when(c < nchunk)
        def _():
            pltpu.sync_copy(idx_hbm.at[c], idx_v)
            pltpu.async_copy(data_hbm.at[idx_v], rows_v, sem).wait()
            pltpu.sync_copy(rows_v, out_hbm.at[pl.ds(c * _CHUNK, _CHUNK)])

        return carry

    lax.fori_loop(0, nj, step, 0)


def _sc_gather(idx, data):
    nchunk = idx.shape[0]
    cin = data.shape[1]
    nj = (nchunk + _NW - 1) // _NW
    mesh = plsc.VectorSubcoreMesh(
        core_axis_name="c", subcore_axis_name="s", num_cores=2, num_subcores=16
    )
    body = functools.partial(_gather_body, nchunk, nj)
    return pl.kernel(
        body,
        out_type=jax.ShapeDtypeStruct((nchunk * _CHUNK, cin), data.dtype),
        mesh=mesh,
        scratch_types=[
            pltpu.VMEM((_CHUNK,), jnp.int32),
            pltpu.VMEM((_CHUNK, cin), data.dtype),
            pltpu.SemaphoreType.DMA,
        ],
    )(idx, data)


def _mm_body(nk, g_ref, w_ref, o_ref):
    k = pl.program_id(1)

    @pl.when(k == 0)
    def _():
        o_ref[...] = jnp.zeros_like(o_ref)

    o_ref[...] += jnp.dot(
        g_ref[0], w_ref[0], preferred_element_type=jnp.float32
    )


def _tc_matmul(g3, wt, n_out, bm):
    nk, _, cin = g3.shape
    cout = wt.shape[2]
    grid = (n_out // bm, nk)
    return pl.pallas_call(
        functools.partial(_mm_body, nk),
        grid=grid,
        in_specs=[
            pl.BlockSpec((1, bm, cin), lambda m, k: (k, m, 0)),
            pl.BlockSpec((1, cin, cout), lambda m, k: (k, 0, 0)),
        ],
        out_specs=pl.BlockSpec((bm, cout), lambda m, k: (m, 0)),
        out_shape=jax.ShapeDtypeStruct((n_out, cout), jnp.float32),
        compiler_params=pltpu.CompilerParams(
            dimension_semantics=("parallel", "arbitrary"),
        ),
    )(g3, wt)


def kernel(data, neighbour, inv_neighbour, weights):
    n, cin = data.shape
    kd = neighbour.shape[1]
    cout = weights.shape[0]

    npad = ((n + _CHUNK - 1) // _CHUNK) * _CHUNK
    nchunk = kd * npad // _CHUNK

    nbr = neighbour.astype(jnp.int32)
    idx = jnp.pad(nbr, ((0, npad - n), (0, 0))).T.reshape(nchunk, _CHUNK)

    gathered = _sc_gather(idx, data)
    g3 = gathered.reshape(kd, npad, cin)
    wt = jnp.transpose(weights, (1, 2, 0))

    bm = 400
    return _tc_matmul(g3, wt, n, bm)


# speedup vs baseline: 1.2223x; 1.0251x over previous
import functools

import jax
import jax.numpy as jnp
from jax import lax
from jax.experimental import pallas as pl
from jax.experimental.pallas import tpu as pltpu
from jax.experimental.pallas import tpu_sc as plsc

_NW = 32
_CHUNK = 128


def _gather_body(nreal, cpw, idx_hbm, data_hbm, out_hbm, idx_v, r0, r1,
                 g0, g1, w0, w1):
    wid = lax.axis_index("s") * 2 + lax.axis_index("c")
    base = wid * cpw
    pltpu.sync_copy(idx_hbm.at[wid], idx_v)

    rows = (r0, r1)
    gsem = (g0, g1)
    wsem = (w0, w1)

    def fire_gather(j, slot):
        pltpu.make_async_copy(
            data_hbm.at[idx_v.at[j]], rows[slot], gsem[slot]
        ).start()

    def wait_gather(j, slot):
        pltpu.make_async_copy(
            data_hbm.at[idx_v.at[j]], rows[slot], gsem[slot]
        ).wait()

    def fire_write(c, slot):
        pltpu.make_async_copy(
            rows[slot], out_hbm.at[pl.ds(c * _CHUNK, _CHUNK)], wsem[slot]
        ).start()

    def wait_write(slot):
        pltpu.make_async_copy(
            rows[slot], out_hbm.at[pl.ds(0, _CHUNK)], wsem[slot]
        ).wait()

    @pl.when(base < nreal)
    def _():
        fire_gather(0, 0)

    def body(i, carry):
        for t in (0, 1):
            j = 2 * i + t
            c = base + j
            slot = t
            nslot = 1 - t
            @pl.when(jnp.logical_and(j >= 1, c - 1 < nreal))
            def _():
                wait_write(nslot)

            @pl.when(jnp.logical_and(j + 1 < cpw, c + 1 < nreal))
            def _():
                fire_gather(j + 1, nslot)

            @pl.when(c < nreal)
            def _():
                wait_gather(j, slot)
                fire_write(c, slot)
        return carry

    lax.fori_loop(0, cpw // 2, body, 0)

    @pl.when(base + cpw - 1 < nreal)
    def _():
        wait_write((cpw - 1) % 2)


def _sc_gather(idx, data, nreal, cpw):
    cin = data.shape[1]
    mesh = plsc.VectorSubcoreMesh(
        core_axis_name="c", subcore_axis_name="s", num_cores=2, num_subcores=16
    )
    body = functools.partial(_gather_body, nreal, cpw)
    return pl.kernel(
        body,
        out_type=jax.ShapeDtypeStruct((nreal * _CHUNK, cin), data.dtype),
        mesh=mesh,
        scratch_types=[
            pltpu.VMEM((cpw, _CHUNK), jnp.int32),
            pltpu.VMEM((_CHUNK, cin), data.dtype),
            pltpu.VMEM((_CHUNK, cin), data.dtype),
            pltpu.SemaphoreType.DMA,
            pltpu.SemaphoreType.DMA,
            pltpu.SemaphoreType.DMA,
            pltpu.SemaphoreType.DMA,
        ],
    )(idx, data)


def _mm_body(g_ref, w_ref, o_ref):
    k = pl.program_id(1)

    @pl.when(k == 0)
    def _():
        o_ref[...] = jnp.zeros_like(o_ref)

    o_ref[...] += jnp.dot(
        g_ref[0], w_ref[k], preferred_element_type=jnp.float32
    )


def _tc_matmul(g3, wt, n_out, bm):
    nk, _, cin = g3.shape
    cout = wt.shape[2]
    grid = (n_out // bm, nk)
    return pl.pallas_call(
        _mm_body,
        grid=grid,
        in_specs=[
            pl.BlockSpec((1, bm, cin), lambda m, k: (k, m, 0)),
            pl.BlockSpec((nk, cin, cout), lambda m, k: (0, 0, 0)),
        ],
        out_specs=pl.BlockSpec((bm, cout), lambda m, k: (m, 0)),
        out_shape=jax.ShapeDtypeStruct((n_out, cout), jnp.float32),
        compiler_params=pltpu.CompilerParams(
            dimension_semantics=("parallel", "arbitrary"),
        ),
    )(g3, wt)


def kernel(data, neighbour, inv_neighbour, weights):
    n, cin = data.shape
    kd = neighbour.shape[1]
    cout = weights.shape[0]

    npad = ((n + _CHUNK - 1) // _CHUNK) * _CHUNK
    nchunk = kd * npad // _CHUNK
    cpw = (nchunk + _NW - 1) // _NW
    cpw = cpw + (cpw % 2)

    nbr = neighbour.astype(jnp.int32)
    idx = jnp.pad(nbr, ((0, npad - n), (0, 0))).T.reshape(nchunk, _CHUNK)
    idx = jnp.pad(idx, ((0, _NW * cpw - nchunk), (0, 0)))
    idx = idx.reshape(_NW, cpw, _CHUNK)

    gathered = _sc_gather(idx, data, nchunk, cpw)
    g3 = gathered.reshape(kd, npad, cin)
    wt = jnp.transpose(weights, (1, 2, 0))

    bm = 400
    return _tc_matmul(g3, wt, n, bm)


# speedup vs baseline: 1.2344x; 1.0099x over previous
import functools

import jax
import jax.numpy as jnp
from jax import lax
from jax.experimental import pallas as pl
from jax.experimental.pallas import tpu as pltpu
from jax.experimental.pallas import tpu_sc as plsc

_NW = 32
_CHUNK = 128


_DEPTH = 6
_AHEAD = 3


def _gather_body(nreal, cpw, idx_hbm, data_hbm, out_hbm, idx_v, *ring):
    wid = lax.axis_index("s") * 2 + lax.axis_index("c")
    base = wid * cpw
    pltpu.sync_copy(idx_hbm.at[wid], idx_v)

    rows = ring[:_DEPTH]
    gsem = ring[_DEPTH:2 * _DEPTH]
    wsem = ring[2 * _DEPTH:]

    def fire_gather(j, slot):
        pltpu.make_async_copy(
            data_hbm.at[idx_v.at[j]], rows[slot], gsem[slot]
        ).start()

    def wait_gather(j, slot):
        pltpu.make_async_copy(
            data_hbm.at[idx_v.at[j]], rows[slot], gsem[slot]
        ).wait()

    def fire_write(c, slot):
        pltpu.make_async_copy(
            rows[slot], out_hbm.at[pl.ds(c * _CHUNK, _CHUNK)], wsem[slot]
        ).start()

    def wait_write(slot):
        pltpu.make_async_copy(
            rows[slot], out_hbm.at[pl.ds(0, _CHUNK)], wsem[slot]
        ).wait()

    for jj in range(_AHEAD):
        @pl.when(base + jj < nreal)
        def _(jj=jj):
            fire_gather(jj, jj)

    def body(i, carry):
        for t in range(_DEPTH):
            j = _DEPTH * i + t
            c = base + j
            aslot = (t + _AHEAD) % _DEPTH
            @pl.when(jnp.logical_and(j + _AHEAD >= _DEPTH,
                                     c + _AHEAD - _DEPTH < nreal))
            def _():
                wait_write(aslot)

            @pl.when(jnp.logical_and(j + _AHEAD < cpw,
                                     c + _AHEAD < nreal))
            def _():
                fire_gather(j + _AHEAD, aslot)

            @pl.when(c < nreal)
            def _():
                wait_gather(j, t)
                fire_write(c, t)
        return carry

    lax.fori_loop(0, cpw // _DEPTH, body, 0)

    for jj in range(cpw - (_DEPTH - _AHEAD), cpw):
        @pl.when(base + jj < nreal)
        def _(jj=jj):
            wait_write(jj % _DEPTH)


def _sc_gather(idx, data, nreal, cpw):
    cin = data.shape[1]
    mesh = plsc.VectorSubcoreMesh(
        core_axis_name="c", subcore_axis_name="s", num_cores=2, num_subcores=16
    )
    body = functools.partial(_gather_body, nreal, cpw)
    return pl.kernel(
        body,
        out_type=jax.ShapeDtypeStruct((nreal * _CHUNK, cin), data.dtype),
        mesh=mesh,
        scratch_types=(
            [pltpu.VMEM((cpw, _CHUNK), jnp.int32)]
            + [pltpu.VMEM((_CHUNK, cin), data.dtype)] * _DEPTH
            + [pltpu.SemaphoreType.DMA] * (2 * _DEPTH)
        ),
    )(idx, data)


def _mm_body(g_ref, w_ref, o_ref):
    k = pl.program_id(1)

    @pl.when(k == 0)
    def _():
        o_ref[...] = jnp.zeros_like(o_ref)

    o_ref[...] += jnp.dot(
        g_ref[0], w_ref[k], preferred_element_type=jnp.float32
    )


def _tc_matmul(g3, wt, n_out, bm):
    nk, _, cin = g3.shape
    cout = wt.shape[2]
    grid = (n_out // bm, nk)
    return pl.pallas_call(
        _mm_body,
        grid=grid,
        in_specs=[
            pl.BlockSpec((1, bm, cin), lambda m, k: (k, m, 0)),
            pl.BlockSpec((nk, cin, cout), lambda m, k: (0, 0, 0)),
        ],
        out_specs=pl.BlockSpec((bm, cout), lambda m, k: (m, 0)),
        out_shape=jax.ShapeDtypeStruct((n_out, cout), jnp.float32),
        compiler_params=pltpu.CompilerParams(
            dimension_semantics=("parallel", "arbitrary"),
        ),
    )(g3, wt)


def kernel(data, neighbour, inv_neighbour, weights):
    n, cin = data.shape
    kd = neighbour.shape[1]
    cout = weights.shape[0]

    npad = ((n + _CHUNK - 1) // _CHUNK) * _CHUNK
    nchunk = kd * npad // _CHUNK
    cpw = (nchunk + _NW - 1) // _NW
    cpw = ((cpw + _DEPTH - 1) // _DEPTH) * _DEPTH

    nbr = neighbour.astype(jnp.int32)
    idx = jnp.pad(nbr, ((0, npad - n), (0, 0))).T.reshape(nchunk, _CHUNK)
    idx = jnp.pad(idx, ((0, _NW * cpw - nchunk), (0, 0)))
    idx = idx.reshape(_NW, cpw, _CHUNK)

    gathered = _sc_gather(idx, data, nchunk, cpw)
    g3 = gathered.reshape(kd, npad, cin)
    wt = jnp.transpose(weights, (1, 2, 0))

    bm = 400
    return _tc_matmul(g3, wt, n, bm)


# speedup vs baseline: 2.3793x; 1.9275x over previous
import functools

import jax
import jax.numpy as jnp
from jax import lax
from jax.experimental import pallas as pl
from jax.experimental.pallas import tpu as pltpu
from jax.experimental.pallas import tpu_sc as plsc

_NW = 32
_CHUNK = 128


_DEPTH = 6
_AHEAD = 3


def _gather_body(nreal, cpw, idx_hbm, data_hbm, out_hbm, idx_v, *ring):
    wid = lax.axis_index("s") * 2 + lax.axis_index("c")
    base = wid * cpw
    pltpu.sync_copy(idx_hbm.at[wid], idx_v)

    rows = ring[:_DEPTH]
    gsem = ring[_DEPTH:2 * _DEPTH]
    wsem = ring[2 * _DEPTH:]

    def fire_gather(j, slot):
        pltpu.make_async_copy(
            data_hbm.at[idx_v.at[j]], rows[slot], gsem[slot]
        ).start()

    def wait_gather(j, slot):
        pltpu.make_async_copy(
            data_hbm.at[idx_v.at[j]], rows[slot], gsem[slot]
        ).wait()

    def fire_write(c, slot):
        pltpu.make_async_copy(
            rows[slot], out_hbm.at[pl.ds(c * _CHUNK, _CHUNK)], wsem[slot]
        ).start()

    def wait_write(slot):
        pltpu.make_async_copy(
            rows[slot], out_hbm.at[pl.ds(0, _CHUNK)], wsem[slot]
        ).wait()

    for jj in range(_AHEAD):
        @pl.when(base + jj < nreal)
        def _(jj=jj):
            fire_gather(jj, jj)

    def body(i, carry):
        for t in range(_DEPTH):
            j = _DEPTH * i + t
            c = base + j
            aslot = (t + _AHEAD) % _DEPTH
            @pl.when(jnp.logical_and(j + _AHEAD >= _DEPTH,
                                     c + _AHEAD - _DEPTH < nreal))
            def _():
                wait_write(aslot)

            @pl.when(jnp.logical_and(j + _AHEAD < cpw,
                                     c + _AHEAD < nreal))
            def _():
                fire_gather(j + _AHEAD, aslot)

            @pl.when(c < nreal)
            def _():
                wait_gather(j, t)
                fire_write(c, t)
        return carry

    lax.fori_loop(0, cpw // _DEPTH, body, 0)

    for jj in range(cpw - (_DEPTH - _AHEAD), cpw):
        @pl.when(base + jj < nreal)
        def _(jj=jj):
            wait_write(jj % _DEPTH)


def _sc_gather(idx, data, nreal, cpw):
    cin = data.shape[1]
    mesh = plsc.VectorSubcoreMesh(
        core_axis_name="c", subcore_axis_name="s", num_cores=2, num_subcores=16
    )
    body = functools.partial(_gather_body, nreal, cpw)
    return pl.kernel(
        body,
        out_type=jax.ShapeDtypeStruct((nreal * _CHUNK, cin), data.dtype),
        mesh=mesh,
        scratch_types=(
            [pltpu.VMEM((cpw, _CHUNK), jnp.int32)]
            + [pltpu.VMEM((_CHUNK, cin), data.dtype)] * _DEPTH
            + [pltpu.SemaphoreType.DMA] * (2 * _DEPTH)
        ),
    )(idx, data)


def _mm_body(nk, g_ref, w_ref, o_ref):
    acc = jnp.zeros(o_ref.shape, jnp.float32)
    for k in range(nk):
        acc += jnp.dot(
            g_ref[k].astype(jnp.bfloat16), w_ref[k],
            preferred_element_type=jnp.float32,
        )
    o_ref[...] = acc


def _tc_matmul(g3, wt, n_out, bm):
    nk, _, cin = g3.shape
    cout = wt.shape[2]
    return pl.pallas_call(
        functools.partial(_mm_body, nk),
        grid=(n_out // bm,),
        in_specs=[
            pl.BlockSpec((nk, bm, cin), lambda m: (0, m, 0)),
            pl.BlockSpec((nk, cin, cout), lambda m: (0, 0, 0)),
        ],
        out_specs=pl.BlockSpec((bm, cout), lambda m: (m, 0)),
        out_shape=jax.ShapeDtypeStruct((n_out, cout), jnp.float32),
        compiler_params=pltpu.CompilerParams(
            dimension_semantics=("arbitrary",),
        ),
    )(g3, wt)


def kernel(data, neighbour, inv_neighbour, weights):
    n, cin = data.shape
    kd = neighbour.shape[1]
    cout = weights.shape[0]

    npad = ((n + _CHUNK - 1) // _CHUNK) * _CHUNK
    nchunk = kd * npad // _CHUNK
    cpw = (nchunk + _NW - 1) // _NW
    cpw = ((cpw + _DEPTH - 1) // _DEPTH) * _DEPTH

    nbr = neighbour.astype(jnp.int32)
    idx = jnp.pad(nbr, ((0, npad - n), (0, 0))).T.reshape(nchunk, _CHUNK)
    idx = jnp.pad(idx, ((0, _NW * cpw - nchunk), (0, 0)))
    idx = idx.reshape(_NW, cpw, _CHUNK)

    gathered = _sc_gather(idx, data, nchunk, cpw)
    g3 = gathered.reshape(kd, npad, cin)
    wt = jnp.transpose(weights, (1, 2, 0)).astype(jnp.bfloat16)

    bm = 400
    return _tc_matmul(g3, wt, n, bm)
